# Initial kernel scaffold; baseline (speedup 1.0000x reference)
import jax
import jax.numpy as jnp
from jax.experimental import pallas as pl


def kernel(b, N, K, L, gpn_obj_ind, gpn_pred_ind, gpn_nrel_ind, gpn_pool_mtx, att_feats, x_pred, fc_feats, att_masks, W1, b1, W2, b2, P1, pb1, P2, pb2):
    raise NotImplementedError("write your pallas kernel here")


# speedup vs baseline: 61.5633x; 61.5633x over previous
import functools

import jax
import jax.numpy as jnp
from jax.experimental import pallas as pl


def _gpn_core(nbatch, seg, nnode, nobj, apad,
              obj_ref, pool_ref, a_ref, w1_ref, b1_ref, w2_ref, b2_ref,
              p1_ref, pb1_ref, p2_ref, pb2_ref, masks_ref,
              loss_ref, score_ref, att_ref, fc_ref, msk_ref, keep_ref):
    f32 = jnp.float32
    gb = 2 * nbatch * seg
    half = nbatch * seg

    obj = obj_ref[...]
    pool = pool_ref[...]
    row = jax.lax.broadcasted_iota(jnp.int32, (gb, 1), 0)
    offs = ((row % half) // seg) * nobj
    lane = jax.lax.broadcasted_iota(jnp.int32, (gb, apad), 1)
    wp = jnp.zeros((gb, apad), f32)
    for n in range(nnode):
        tgt = obj[:, n:n + 1] + offs
        wp = wp + jnp.where(lane == tgt, pool[:, n:n + 1], 0.0)

    aw = jax.lax.dot_general(a_ref[...], w1_ref[...],
                             (((1,), (1,)), ((), ())),
                             preferred_element_type=f32)
    h = jnp.dot(wp, aw, preferred_element_type=f32) + b1_ref[...]
    h = jnp.maximum(h, 0.0)
    z = jnp.sum(h * w2_ref[...], axis=1, keepdims=True)
    score = jax.nn.sigmoid(z + b2_ref[0, 0])

    logp = jnp.maximum(jnp.log(score), -100.0)
    log1m = jnp.maximum(jnp.log(1.0 - score), -100.0)
    contrib = jnp.where(row < half, logp, log1m)
    loss_ref[...] = jnp.full((1, 128), -jnp.sum(contrib) / gb, f32)

    s400 = jnp.concatenate([score[0:seg], score[half:half + seg]], axis=0)
    i400 = jax.lax.broadcasted_iota(jnp.int32, (2 * seg, 1), 0)
    m = jnp.max(s400)
    r = jnp.max(jnp.where(s400 == m, i400, -1))
    keep_ref[...] = jnp.full((1, 128), r, jnp.int32)
    score_ref[...] = jnp.full((1, 128), m, f32)

    g_idx = jnp.where(r < seg, r, r + (nbatch - 1) * seg)
    pool_row = pool_ref[pl.ds(g_idx, 1), :]
    obj_row = obj_ref[pl.ds(g_idx, 1), :]
    msk_ref[...] = masks_ref[pl.ds(r, 1), :]
    acc = jnp.zeros((1, a_ref.shape[1]), f32)
    for n in range(nnode):
        o = obj_row[0, n]
        rown = a_ref[pl.ds(o, 1), :]
        att_ref[n:n + 1, :] = rown
        acc = acc + pool_row[0, n] * rown
    fc1 = jax.lax.dot_general(acc, p1_ref[...], (((1,), (1,)), ((), ())),
                              preferred_element_type=f32) + pb1_ref[...]
    fc2 = jax.lax.dot_general(fc1, p2_ref[...], (((1,), (1,)), ((), ())),
                              preferred_element_type=f32) + pb2_ref[...]
    fc_ref[...] = fc2


def kernel(b, N, K, L, gpn_obj_ind, gpn_pred_ind, gpn_nrel_ind, gpn_pool_mtx,
           att_feats, x_pred, fc_feats, att_masks, W1, b1, W2, b2, P1, pb1,
           P2, pb2):
    nbatch, _, seg, nnode = gpn_obj_ind.shape
    nobj = att_feats.shape[1]
    feat = att_feats.shape[2]
    hid = W1.shape[0]
    gb = 2 * nbatch * seg
    stacked = nbatch * nobj
    apad = -(-stacked // 128) * 128

    obj2 = jnp.transpose(gpn_obj_ind, (1, 0, 2, 3)).reshape(gb, nnode)
    obj2 = jnp.pad(obj2.astype(jnp.int32), ((0, 0), (0, 128 - nnode)))
    pool2 = jnp.pad(jnp.transpose(gpn_pool_mtx, (1, 0, 2, 3)).reshape(gb, nnode),
                    ((0, 0), (0, 128 - nnode)))
    a_stack = jnp.pad(att_feats.reshape(stacked, feat),
                      ((0, apad - stacked), (0, 0)))
    masks0 = jnp.pad(att_masks[0].reshape(2 * seg, nnode),
                     ((0, 0), (0, 128 - nnode)))

    core = functools.partial(_gpn_core, nbatch, seg, nnode, nobj, apad)
    outs = pl.pallas_call(
        core,
        out_shape=[
            jax.ShapeDtypeStruct((1, 128), jnp.float32),
            jax.ShapeDtypeStruct((1, 128), jnp.float32),
            jax.ShapeDtypeStruct((nnode, feat), jnp.float32),
            jax.ShapeDtypeStruct((1, feat), jnp.float32),
            jax.ShapeDtypeStruct((1, 128), jnp.float32),
            jax.ShapeDtypeStruct((1, 128), jnp.int32),
        ],
    )(obj2, pool2, a_stack, W1, b1.reshape(1, hid), W2, b2.reshape(1, 1),
      P1, pb1.reshape(1, hid), P2, pb2.reshape(1, feat), masks0)

    o_loss, o_score, o_att, o_fc, o_msk, o_keep = outs
    return (o_loss[0, 0], o_score[0, 0:1], o_att[None], o_fc,
            o_msk[:, 0:nnode], o_keep[0, 0:1])
